# Initial kernel scaffold; baseline (speedup 1.0000x reference)
import jax
import jax.numpy as jnp
from jax.experimental import pallas as pl


def kernel(x, edge_index, batch, W1, b1, W2, b2, W3, b3, W4, b4, lin1_W, lin1_b, lin2_W, lin2_b):
    raise NotImplementedError("write your pallas kernel here")


# speedup vs baseline: 8.3548x; 8.3548x over previous
import functools

import jax
import jax.numpy as jnp
from jax import lax
from jax.experimental import pallas as pl
from jax.experimental.pallas import tpu as pltpu
from jax.experimental.pallas import tpu_sc as plsc

N = 10000
E = 160000
D = 256
H = 512
G = 64

NC, NS = 2, 16
NP = 10240
C = 4
FC = H // C
RPT = NP // NS
KW = 128
EP = 163840
EPT = EP // NS
NW = EPT // KW

BN = 1024
NB = NP // BN


def _agg_body(ch, g_hbm, srcw, dstw, a_hbm, stile, dtile, rows, acc, sem):
    cid = lax.axis_index("c")
    sid = lax.axis_index("s")

    pltpu.sync_copy(srcw.at[sid], stile)
    pltpu.sync_copy(dstw.at[sid], dtile)

    for step in range(ch // NC):
        c = step * NC + cid
        pltpu.sync_copy(g_hbm.at[c].at[pl.ds(sid * RPT, RPT)],
                        acc.at[pl.ds(sid * RPT, RPT)])
        plsc.subcore_barrier()

        def w(i, carry):
            pltpu.async_copy(g_hbm.at[c].at[stile.at[i]], rows, sem).wait()
            pltpu.sync_copy(rows, acc.at[dtile.at[i]], add=True)
            return carry
        lax.fori_loop(0, NW, w, None)
        plsc.subcore_barrier()

        pltpu.sync_copy(acc.at[pl.ds(sid * RPT, RPT)],
                        a_hbm.at[c].at[pl.ds(sid * RPT, RPT)])
        plsc.subcore_barrier()


@functools.cache
def _agg_call(ch):
    mesh = plsc.VectorSubcoreMesh(core_axis_name="c", subcore_axis_name="s",
                                  num_cores=NC, num_subcores=NS)
    return pl.kernel(
        functools.partial(_agg_body, ch),
        out_type=jax.ShapeDtypeStruct((ch, NP, FC), jnp.float32),
        mesh=mesh,
        scratch_types=[
            pltpu.VMEM((NW, KW), jnp.int32),
            pltpu.VMEM((NW, KW), jnp.int32),
            pltpu.VMEM((KW, FC), jnp.float32),
            pltpu.VMEM_SHARED((NP, FC), jnp.float32),
            pltpu.SemaphoreType.DMA,
        ],
    )


def _dinv(deg_ref):
    return lax.rsqrt(deg_ref[:, :1])


def _mm1_body(x_ref, deg_ref, w_ref, g_ref):
    dinv = _dinv(deg_ref)
    gv = jnp.dot(x_ref[...], w_ref[...],
                 preferred_element_type=jnp.float32,
                 precision=lax.Precision.HIGHEST) * dinv
    for c in range(C):
        g_ref[c] = gv[:, c * FC:(c + 1) * FC]


_mm1_call = pl.pallas_call(
    _mm1_body,
    grid=(NB,),
    in_specs=[
        pl.BlockSpec((BN, D), lambda i: (i, 0)),
        pl.BlockSpec((BN, FC), lambda i: (i, 0)),
        pl.BlockSpec((D, H), lambda i: (0, 0)),
    ],
    out_specs=pl.BlockSpec((C, BN, FC), lambda i: (0, i, 0)),
    out_shape=jax.ShapeDtypeStruct((C, NP, FC), jnp.float32),
)


def _mml_body(a_ref, deg_ref, w_ref, b_ref, g_ref):
    dinv = _dinv(deg_ref)
    hcat = jnp.concatenate([a_ref[c] for c in range(C)], axis=-1)
    h = jnp.maximum(hcat * dinv + b_ref[...], 0.0)
    gv = jnp.dot(h, w_ref[...], preferred_element_type=jnp.float32,
                 precision=lax.Precision.HIGHEST) * dinv
    for c in range(C):
        g_ref[c] = gv[:, c * FC:(c + 1) * FC]


_mml_call = pl.pallas_call(
    _mml_body,
    grid=(NB,),
    in_specs=[
        pl.BlockSpec((C, BN, FC), lambda i: (0, i, 0)),
        pl.BlockSpec((BN, FC), lambda i: (i, 0)),
        pl.BlockSpec((H, H), lambda i: (0, 0)),
        pl.BlockSpec((1, H), lambda i: (0, 0)),
    ],
    out_specs=pl.BlockSpec((C, BN, FC), lambda i: (0, i, 0)),
    out_shape=jax.ShapeDtypeStruct((C, NP, FC), jnp.float32),
)


def _final_body(a_ref, deg_ref, b4_ref, bat_ref,
                w1_ref, b1_ref, w2_ref, b2_ref, out_ref, pool):
    i = pl.program_id(0)

    @pl.when(i == 0)
    def _():
        pool[...] = jnp.full((G, H), -jnp.inf, jnp.float32)

    dinv = _dinv(deg_ref)
    hcat = jnp.concatenate([a_ref[c] for c in range(C)], axis=-1)
    h = jnp.maximum(hcat * dinv + b4_ref[...], 0.0)
    bb = bat_ref[...]
    glo = jnp.min(bb)
    ghi = jnp.minimum(jnp.max(bb), G - 1)

    def gbody(g, carry):
        @pl.when((g >= glo) & (g <= ghi))
        def _():
            m = bb == g
            contrib = jnp.max(jnp.where(m, h, -jnp.inf), axis=0, keepdims=True)
            pool[pl.ds(g, 1), :] = jnp.maximum(pool[pl.ds(g, 1), :], contrib)
        return carry
    lax.fori_loop(0, G, gbody, None)

    @pl.when(i == NB - 1)
    def _():
        p = pool[...]
        p = jnp.maximum(
            jnp.dot(p, w1_ref[...], preferred_element_type=jnp.float32,
                 precision=lax.Precision.HIGHEST)
            + b1_ref[...], 0.0)
        out_ref[...] = (jnp.dot(p, w2_ref[...],
                                preferred_element_type=jnp.float32,
                 precision=lax.Precision.HIGHEST)
                        + b2_ref[...])


_final_call = pl.pallas_call(
    _final_body,
    grid=(NB,),
    in_specs=[
        pl.BlockSpec((C, BN, FC), lambda i: (0, i, 0)),
        pl.BlockSpec((BN, FC), lambda i: (i, 0)),
        pl.BlockSpec((1, H), lambda i: (0, 0)),
        pl.BlockSpec((BN, 1), lambda i: (i, 0)),
        pl.BlockSpec((H, H), lambda i: (0, 0)),
        pl.BlockSpec((1, H), lambda i: (0, 0)),
        pl.BlockSpec((H, 1), lambda i: (0, 0)),
        pl.BlockSpec((1, 1), lambda i: (0, 0)),
    ],
    out_specs=pl.BlockSpec((G, 1), lambda i: (0, 0)),
    out_shape=jax.ShapeDtypeStruct((G, 1), jnp.float32),
    scratch_shapes=[pltpu.VMEM((G, H), jnp.float32)],
)


def kernel(x, edge_index, batch, W1, b1, W2, b2, W3, b3, W4, b4,
           lin1_W, lin1_b, lin2_W, lin2_b):
    src = edge_index[0]
    dst = edge_index[1]
    npad = EP - E
    pad_src = (jnp.arange(npad, dtype=jnp.int32) * 37) % N
    pad_dst = N + (jnp.arange(npad, dtype=jnp.int32) % (NP - N))
    src_w = jnp.concatenate([src, pad_src]).reshape(NS, NW, KW)
    dst_w = jnp.concatenate([dst, pad_dst]).reshape(NS, NW, KW)

    x_p = jnp.zeros((NP, D), jnp.float32).at[:N].set(x)
    batch_p = jnp.concatenate(
        [batch, jnp.full((NP - N,), G, jnp.int32)]).reshape(NP, 1)

    ones2 = jnp.ones((NC, NP, FC), jnp.float32)
    deg = _agg_call(NC)(ones2, src_w, dst_w)[0]

    agg = _agg_call(C)
    g = _mm1_call(x_p, deg, W1)
    a = agg(g, src_w, dst_w)
    g = _mml_call(a, deg, W2, b1.reshape(1, H))
    a = agg(g, src_w, dst_w)
    g = _mml_call(a, deg, W3, b2.reshape(1, H))
    a = agg(g, src_w, dst_w)
    g = _mml_call(a, deg, W4, b3.reshape(1, H))
    a = agg(g, src_w, dst_w)

    return _final_call(a, deg, b4.reshape(1, H), batch_p,
                       lin1_W, lin1_b.reshape(1, H),
                       lin2_W, lin2_b.reshape(1, 1))


# speedup vs baseline: 10.8153x; 1.2945x over previous
import functools

import jax
import jax.numpy as jnp
from jax import lax
from jax.experimental import pallas as pl
from jax.experimental.pallas import tpu as pltpu
from jax.experimental.pallas import tpu_sc as plsc

N = 10000
E = 160000
D = 256
H = 512
G = 64

NC, NS = 2, 16
NP = 10240
C = 4
FC = H // C
RPT = NP // NS
KW = 128
EP = 163840
EPT = EP // NS
NW = EPT // KW
DB = 16

BN = 1024
NB = NP // BN


def _agg_body(ch, g_hbm, srcw, dstw, a_hbm, stile, dtile, rows0, rows1,
              acc, sem0, sem1):
    cid = lax.axis_index("c")
    sid = lax.axis_index("s")

    pltpu.sync_copy(srcw.at[sid], stile)

    for step in range(ch // NC):
        c = step * NC + cid
        pltpu.sync_copy(g_hbm.at[c].at[pl.ds(sid * RPT, RPT)],
                        acc.at[pl.ds(sid * RPT, RPT)])
        plsc.subcore_barrier()

        pltpu.async_copy(g_hbm.at[c].at[stile.at[0]], rows0, sem0)

        def blk(b, carry):
            pltpu.sync_copy(dstw.at[sid].at[pl.ds(b * DB, DB)], dtile)

            def pair(j, carry2):
                w0 = b * DB + 2 * j
                pltpu.make_async_copy(
                    g_hbm.at[c].at[stile.at[w0]], rows0, sem0).wait()
                pltpu.async_copy(
                    g_hbm.at[c].at[stile.at[w0 + 1]], rows1, sem1)
                pltpu.sync_copy(rows0, acc.at[dtile.at[2 * j]], add=True)
                pltpu.make_async_copy(
                    g_hbm.at[c].at[stile.at[w0 + 1]], rows1, sem1).wait()
                nxt = jnp.minimum(w0 + 2, NW - 1)
                pltpu.async_copy(g_hbm.at[c].at[stile.at[nxt]], rows0, sem0)
                pltpu.sync_copy(rows1, acc.at[dtile.at[2 * j + 1]], add=True)
                return carry2
            lax.fori_loop(0, DB // 2, pair, None)
            return carry
        lax.fori_loop(0, NW // DB, blk, None)
        pltpu.make_async_copy(
            g_hbm.at[c].at[stile.at[NW - 1]], rows0, sem0).wait()
        plsc.subcore_barrier()

        pltpu.sync_copy(acc.at[pl.ds(sid * RPT, RPT)],
                        a_hbm.at[c].at[pl.ds(sid * RPT, RPT)])
        plsc.subcore_barrier()


@functools.cache
def _agg_call(ch):
    mesh = plsc.VectorSubcoreMesh(core_axis_name="c", subcore_axis_name="s",
                                  num_cores=NC, num_subcores=NS)
    return pl.kernel(
        functools.partial(_agg_body, ch),
        out_type=jax.ShapeDtypeStruct((ch, NP, FC), jnp.float32),
        mesh=mesh,
        scratch_types=[
            pltpu.VMEM((NW, KW), jnp.int32),
            pltpu.VMEM((DB, KW), jnp.int32),
            pltpu.VMEM((KW, FC), jnp.float32),
            pltpu.VMEM((KW, FC), jnp.float32),
            pltpu.VMEM_SHARED((NP, FC), jnp.float32),
            pltpu.SemaphoreType.DMA,
            pltpu.SemaphoreType.DMA,
        ],
    )


def _xdot(a, b):
    return jnp.dot(a, b, preferred_element_type=jnp.float32,
                   precision=lax.Precision.HIGHEST)


def _dinv(deg_ref):
    deg = deg_ref[:, :1]
    y = lax.rsqrt(deg)
    return y * (1.5 - 0.5 * deg * y * y)


def _mm1_body(x_ref, deg_ref, w_ref, g_ref):
    dinv = _dinv(deg_ref)
    gv = _xdot(x_ref[...], w_ref[...]) * dinv
    for c in range(C):
        g_ref[c] = gv[:, c * FC:(c + 1) * FC]


_mm1_call = pl.pallas_call(
    _mm1_body,
    grid=(NB,),
    in_specs=[
        pl.BlockSpec((BN, D), lambda i: (i, 0)),
        pl.BlockSpec((BN, FC), lambda i: (i, 0)),
        pl.BlockSpec((D, H), lambda i: (0, 0)),
    ],
    out_specs=pl.BlockSpec((C, BN, FC), lambda i: (0, i, 0)),
    out_shape=jax.ShapeDtypeStruct((C, NP, FC), jnp.float32),
)


def _mml_body(a_ref, deg_ref, w_ref, b_ref, g_ref):
    dinv = _dinv(deg_ref)
    hcat = jnp.concatenate([a_ref[c] for c in range(C)], axis=-1)
    h = jnp.maximum(hcat * dinv + b_ref[...], 0.0)
    gv = _xdot(h, w_ref[...]) * dinv
    for c in range(C):
        g_ref[c] = gv[:, c * FC:(c + 1) * FC]


_mml_call = pl.pallas_call(
    _mml_body,
    grid=(NB,),
    in_specs=[
        pl.BlockSpec((C, BN, FC), lambda i: (0, i, 0)),
        pl.BlockSpec((BN, FC), lambda i: (i, 0)),
        pl.BlockSpec((H, H), lambda i: (0, 0)),
        pl.BlockSpec((1, H), lambda i: (0, 0)),
    ],
    out_specs=pl.BlockSpec((C, BN, FC), lambda i: (0, i, 0)),
    out_shape=jax.ShapeDtypeStruct((C, NP, FC), jnp.float32),
)


def _final_body(a_ref, deg_ref, b4_ref, bat_ref,
                w1_ref, b1_ref, w2_ref, b2_ref, out_ref, pool):
    i = pl.program_id(0)

    @pl.when(i == 0)
    def _():
        pool[...] = jnp.full((G, H), -jnp.inf, jnp.float32)

    dinv = _dinv(deg_ref)
    hcat = jnp.concatenate([a_ref[c] for c in range(C)], axis=-1)
    h = jnp.maximum(hcat * dinv + b4_ref[...], 0.0)
    bb = bat_ref[...]
    glo = jnp.min(bb)
    ghi = jnp.minimum(jnp.max(bb), G - 1)

    def gbody(g, carry):
        @pl.when((g >= glo) & (g <= ghi))
        def _():
            m = bb == g
            contrib = jnp.max(jnp.where(m, h, -jnp.inf), axis=0, keepdims=True)
            pool[pl.ds(g, 1), :] = jnp.maximum(pool[pl.ds(g, 1), :], contrib)
        return carry
    lax.fori_loop(0, G, gbody, None)

    @pl.when(i == NB - 1)
    def _():
        p = pool[...]
        p = jnp.maximum(
            _xdot(p, w1_ref[...]) + b1_ref[...], 0.0)
        out_ref[...] = _xdot(p, w2_ref[...]) + b2_ref[...]


_final_call = pl.pallas_call(
    _final_body,
    grid=(NB,),
    in_specs=[
        pl.BlockSpec((C, BN, FC), lambda i: (0, i, 0)),
        pl.BlockSpec((BN, FC), lambda i: (i, 0)),
        pl.BlockSpec((1, H), lambda i: (0, 0)),
        pl.BlockSpec((BN, 1), lambda i: (i, 0)),
        pl.BlockSpec((H, H), lambda i: (0, 0)),
        pl.BlockSpec((1, H), lambda i: (0, 0)),
        pl.BlockSpec((H, 1), lambda i: (0, 0)),
        pl.BlockSpec((1, 1), lambda i: (0, 0)),
    ],
    out_specs=pl.BlockSpec((G, 1), lambda i: (0, 0)),
    out_shape=jax.ShapeDtypeStruct((G, 1), jnp.float32),
    scratch_shapes=[pltpu.VMEM((G, H), jnp.float32)],
)


def kernel(x, edge_index, batch, W1, b1, W2, b2, W3, b3, W4, b4,
           lin1_W, lin1_b, lin2_W, lin2_b):
    src = edge_index[0]
    dst = edge_index[1]
    npad = EP - E
    pad_src = (jnp.arange(npad, dtype=jnp.int32) * 37) % N
    pad_dst = N + (jnp.arange(npad, dtype=jnp.int32) % (NP - N))
    src_w = jnp.concatenate([src, pad_src]).reshape(NS, NW, KW)
    dst_w = jnp.concatenate([dst, pad_dst]).reshape(NS, NW, KW)

    x_p = jnp.zeros((NP, D), jnp.float32).at[:N].set(x)
    batch_p = jnp.concatenate(
        [batch, jnp.full((NP - N,), G, jnp.int32)]).reshape(NP, 1)

    ones2 = jnp.ones((NC, NP, FC), jnp.float32)
    deg = _agg_call(NC)(ones2, src_w, dst_w)[0]

    agg = _agg_call(C)
    g = _mm1_call(x_p, deg, W1)
    a = agg(g, src_w, dst_w)
    g = _mml_call(a, deg, W2, b1.reshape(1, H))
    a = agg(g, src_w, dst_w)
    g = _mml_call(a, deg, W3, b2.reshape(1, H))
    a = agg(g, src_w, dst_w)
    g = _mml_call(a, deg, W4, b3.reshape(1, H))
    a = agg(g, src_w, dst_w)

    return _final_call(a, deg, b4.reshape(1, H), batch_p,
                       lin1_W, lin1_b.reshape(1, H),
                       lin2_W, lin2_b.reshape(1, 1))
